# baseline (speedup 1.0000x reference)
import functools

import jax
import jax.numpy as jnp
from jax import lax
from jax.experimental import pallas as pl
from jax.experimental.pallas import tpu as pltpu
from jax.experimental.pallas import tpu_sc as plsc

_NC = 2
_NS = 16
_NW = _NC * _NS
_CHUNK = 128


@functools.lru_cache(maxsize=None)
def _make_gather(num_idx: int, vocab: int, embed: int):
    assert num_idx % (_NW * _CHUNK) == 0
    n_per_w = num_idx // _NW
    n_ch = n_per_w // _CHUNK

    mesh = plsc.VectorSubcoreMesh(core_axis_name="c", subcore_axis_name="s")

    @functools.partial(
        pl.kernel,
        out_type=jax.ShapeDtypeStruct((num_idx, embed), jnp.float32),
        mesh=mesh,
        scratch_types=[
            pltpu.VMEM((n_ch, _CHUNK), jnp.int32),
            pltpu.VMEM((n_per_w, embed), jnp.float32),
            pltpu.SemaphoreType.DMA,
        ],
    )
    def gather_kernel(idx_hbm, table_hbm, out_hbm, idx_v, rows_v, sem):
        wid = lax.axis_index("s") * _NC + lax.axis_index("c")
        pltpu.sync_copy(idx_hbm.at[pl.ds(wid * n_ch, n_ch)], idx_v)
        copies = [
            pltpu.async_copy(
                table_hbm.at[idx_v.at[j]],
                rows_v.at[pl.ds(j * _CHUNK, _CHUNK)],
                sem,
            )
            for j in range(n_ch)
        ]
        for c in copies:
            c.wait()
        pltpu.sync_copy(rows_v, out_hbm.at[pl.ds(wid * n_per_w, n_per_w)])

    return gather_kernel


@functools.lru_cache(maxsize=None)
def _make_dense_half(batch: int, seq: int, seq_half: int, embed: int,
                     hidden: int, t_blk: int, off_blk: int, aliased: bool):
    assert seq_half % t_blk == 0
    grid = (seq_half // t_blk, batch)

    def compute(x_ref, w_ref, b_ref, p_ref, g_ref, be_ref, o_ref):
        x = x_ref[0]
        y = jnp.dot(x, w_ref[...], preferred_element_type=jnp.float32)
        y = y + b_ref[...] + p_ref[...]
        mean = jnp.mean(y, axis=-1, keepdims=True)
        yc = y - mean
        var = jnp.mean(yc * yc, axis=-1, keepdims=True)
        o_ref[0] = (g_ref[...] * lax.rsqrt(var + 1e-6)) * yc + be_ref[...]

    if aliased:
        def body(prev_ref, x_ref, w_ref, b_ref, p_ref, g_ref, be_ref, o_ref):
            del prev_ref
            compute(x_ref, w_ref, b_ref, p_ref, g_ref, be_ref, o_ref)
    else:
        body = compute

    in_specs = [
        pl.BlockSpec((1, t_blk, embed), lambda j, i: (i, j, 0)),
        pl.BlockSpec((embed, hidden), lambda j, i: (0, 0)),
        pl.BlockSpec((1, hidden), lambda j, i: (0, 0)),
        pl.BlockSpec((t_blk, hidden), lambda j, i: (j + off_blk, 0)),
        pl.BlockSpec((1, hidden), lambda j, i: (0, 0)),
        pl.BlockSpec((1, hidden), lambda j, i: (0, 0)),
    ]
    kwargs = {}
    if aliased:
        in_specs = [pl.BlockSpec(memory_space=pl.ANY)] + in_specs
        kwargs["input_output_aliases"] = {0: 0}

    return pl.pallas_call(
        body,
        grid=grid,
        in_specs=in_specs,
        out_specs=pl.BlockSpec((1, t_blk, hidden),
                               lambda j, i: (i, j + off_blk, 0)),
        out_shape=jax.ShapeDtypeStruct((batch, seq, hidden), jnp.float32),
        **kwargs,
    )


def kernel(sequence, token_table, W, b, pos_table, gamma, beta):
    batch, seq = sequence.shape
    vocab, embed = token_table.shape
    hidden = W.shape[1]
    half = seq // 2
    t_blk = 1024
    nblk_half = half // t_blk

    seq32 = sequence.astype(jnp.int32)
    idx_a = seq32[:, :half].reshape(-1, _CHUNK)
    idx_b = seq32[:, half:].reshape(-1, _CHUNK)
    gath = _make_gather(batch * half, vocab, embed)
    g_a = gath(idx_a, token_table).reshape(batch, half, embed)
    g_b = gath(idx_b, token_table).reshape(batch, half, embed)

    b2 = b.reshape(1, hidden)
    g2 = gamma.reshape(1, hidden)
    be2 = beta.reshape(1, hidden)
    d1 = _make_dense_half(batch, seq, half, embed, hidden, t_blk, 0, False)
    d2 = _make_dense_half(batch, seq, half, embed, hidden, t_blk,
                          nblk_half, True)
    o1 = d1(g_a, W, b2, pos_table, g2, be2)
    return d2(o1, g_b, W, b2, pos_table, g2, be2)


# speedup vs baseline: 1.0580x; 1.0580x over previous
import functools

import jax
import jax.numpy as jnp
from jax import lax
from jax.experimental import pallas as pl
from jax.experimental.pallas import tpu as pltpu
from jax.experimental.pallas import tpu_sc as plsc

_NC = 2
_NS = 16
_NW = _NC * _NS
_CHUNK = 128


@functools.lru_cache(maxsize=None)
def _make_gather(batch: int, seq: int, vocab: int, embed: int):
    num_idx = batch * seq
    assert num_idx % (_NW * _CHUNK) == 0
    n_per_w = num_idx // _NW
    n_ch = n_per_w // _CHUNK
    assert seq % n_per_w == 0
    w_per_row = seq // n_per_w

    mesh = plsc.VectorSubcoreMesh(core_axis_name="c", subcore_axis_name="s")

    @functools.partial(
        pl.kernel,
        out_type=jax.ShapeDtypeStruct((num_idx, embed), jnp.float32),
        mesh=mesh,
        scratch_types=[
            pltpu.VMEM((n_ch, _CHUNK), jnp.int32),
            pltpu.VMEM((n_per_w, embed), jnp.float32),
            pltpu.SemaphoreType.DMA,
        ],
    )
    def gather_kernel(seq_hbm, table_hbm, out_hbm, idx_v, rows_v, sem):
        wid = lax.axis_index("s") * _NC + lax.axis_index("c")
        row = wid // w_per_row
        col0 = (wid % w_per_row) * n_per_w
        for j in range(n_ch):
            pltpu.sync_copy(
                seq_hbm.at[row, pl.ds(col0 + j * _CHUNK, _CHUNK)],
                idx_v.at[j],
            )
        copies = [
            pltpu.async_copy(
                table_hbm.at[idx_v.at[j]],
                rows_v.at[pl.ds(j * _CHUNK, _CHUNK)],
                sem,
            )
            for j in range(n_ch)
        ]
        for c in copies:
            c.wait()
        pltpu.sync_copy(rows_v, out_hbm.at[pl.ds(wid * n_per_w, n_per_w)])

    return gather_kernel


@functools.lru_cache(maxsize=None)
def _make_dense(batch: int, seq: int, embed: int, hidden: int, t_blk: int):
    assert seq % t_blk == 0
    sblk = seq // t_blk
    grid = (sblk, batch)

    def body(x_ref, w_ref, b_ref, p_ref, g_ref, be_ref, o_ref):
        x = x_ref[...].astype(jnp.bfloat16)
        y = jnp.dot(x, w_ref[...], preferred_element_type=jnp.float32)
        y = y + b_ref[...] + p_ref[...]
        mean = jnp.mean(y, axis=-1, keepdims=True)
        yc = y - mean
        var = jnp.mean(yc * yc, axis=-1, keepdims=True)
        o_ref[0] = (g_ref[...] * lax.rsqrt(var + 1e-6)) * yc + be_ref[...]

    return pl.pallas_call(
        body,
        grid=grid,
        in_specs=[
            pl.BlockSpec((t_blk, embed), lambda j, i: (i * sblk + j, 0)),
            pl.BlockSpec((embed, hidden), lambda j, i: (0, 0)),
            pl.BlockSpec((hidden,), lambda j, i: (0,)),
            pl.BlockSpec((t_blk, hidden), lambda j, i: (j, 0)),
            pl.BlockSpec((hidden,), lambda j, i: (0,)),
            pl.BlockSpec((hidden,), lambda j, i: (0,)),
        ],
        out_specs=pl.BlockSpec((1, t_blk, hidden), lambda j, i: (i, j, 0)),
        out_shape=jax.ShapeDtypeStruct((batch, seq, hidden), jnp.float32),
    )


def kernel(sequence, token_table, W, b, pos_table, gamma, beta):
    batch, seq = sequence.shape
    vocab, embed = token_table.shape
    hidden = W.shape[1]

    gathered = _make_gather(batch, seq, vocab, embed)(
        sequence.astype(jnp.int32), token_table
    )
    dense = _make_dense(batch, seq, embed, hidden, 2048)
    return dense(
        gathered,
        W.astype(jnp.bfloat16),
        b,
        pos_table[:seq],
        gamma,
        beta,
    )
